# baseline (speedup 1.0000x reference)
import functools

import jax
import jax.numpy as jnp
from jax.experimental import pallas as pl
from jax.experimental.pallas import tpu as pltpu

_SMEM = getattr(pltpu, "SMEM", None)
if _SMEM is None:
    _SMEM = pltpu.MemorySpace.SMEM

_BN = 512
_BE = 2048


def _lrelu(v):
    return jnp.where(v >= 0, v, 0.2 * v)


def _elu(v):
    return jnp.where(v > 0, v, jnp.exp(v) - 1.0)


def _embed_kernel(ids_ref, emb_ref, out_ref):
    def body(i, c):
        out_ref[pl.ds(i, 1), :] = emb_ref[pl.ds(ids_ref[0, 0, i], 1), :]
        return c
    jax.lax.fori_loop(0, out_ref.shape[0], body, 0)


def _dense1_kernel(h0_ref, w_ref, asrc_ref, adst_ref, hw_ref, al_ref, ar_ref):
    hw = jnp.dot(h0_ref[...], w_ref[...], preferred_element_type=jnp.float32)
    hw_ref[...] = hw
    heads = asrc_ref.shape[0]
    ch = asrc_ref.shape[1]
    als = []
    ars = []
    for h in range(heads):
        blk = hw[:, h * ch:(h + 1) * ch]
        als.append((blk * asrc_ref[pl.ds(h, 1), :]).sum(axis=1, keepdims=True))
        ars.append((blk * adst_ref[pl.ds(h, 1), :]).sum(axis=1, keepdims=True))
    al_ref[...] = jnp.concatenate(als, axis=1)
    ar_ref[...] = jnp.concatenate(ars, axis=1)


def _den_kernel(se_ref, de_ref, al_ref, ar_ref, den_ref):
    @pl.when(pl.program_id(0) == 0)
    def _():
        den_ref[...] = jnp.zeros_like(den_ref)

    lane = jax.lax.broadcasted_iota(jnp.int32, (1, 1, 128), 2)

    def body(i, c):
        s = se_ref[0, 0, i]
        d = de_ref[0, 0, i]
        rs, cs = s // 128, s % 128
        rd, cd = d // 128, d % 128
        ms = (lane == cs).astype(jnp.float32)
        md = (lane == cd).astype(jnp.float32)
        al = jnp.sum(al_ref[:, pl.ds(rs, 1), :] * ms, axis=2, keepdims=True)
        ar = jnp.sum(ar_ref[:, pl.ds(rd, 1), :] * md, axis=2, keepdims=True)
        den_ref[:, pl.ds(rd, 1), :] += jnp.exp(_lrelu(al + ar)) * md
        return c
    jax.lax.fori_loop(0, se_ref.shape[2], body, 0)


def _agg_kernel(h_start, hpc, se_ref, de_ref, al_ref, ar_ref, den_ref, hw_ref, out_ref):
    cw = out_ref.shape[1]
    ch = cw // hpc

    @pl.when(pl.program_id(0) == 0)
    def _():
        out_ref[...] = jnp.zeros(out_ref.shape, out_ref.dtype)

    lane = jax.lax.broadcasted_iota(jnp.int32, (1, 1, 128), 2)

    def body(i, carry):
        s = se_ref[0, 0, i]
        d = de_ref[0, 0, i]
        rs, cs = s // 128, s % 128
        rd, cd = d // 128, d % 128
        ms = (lane == cs).astype(jnp.float32)
        md = (lane == cd).astype(jnp.float32)
        al = jnp.sum(al_ref[pl.ds(h_start, hpc), pl.ds(rs, 1), :] * ms,
                     axis=2, keepdims=True)
        ar = jnp.sum(ar_ref[pl.ds(h_start, hpc), pl.ds(rd, 1), :] * md,
                     axis=2, keepdims=True)
        den = jnp.sum(den_ref[pl.ds(h_start, hpc), pl.ds(rd, 1), :] * md,
                      axis=2, keepdims=True)
        att = jnp.exp(_lrelu(al + ar)) / (den + 1e-16)
        parts = [jnp.broadcast_to(att[h:h + 1, 0, :], (1, ch)) for h in range(hpc)]
        m = parts[0] if hpc == 1 else jnp.concatenate(parts, axis=1)
        out_ref[pl.ds(d, 1), :] += m * hw_ref[pl.ds(s, 1), :]
        return carry
    jax.lax.fori_loop(0, se_ref.shape[2], body, 0)


def _dense2_kernel(agg_ref, b1_ref, w_ref, asrc_ref, adst_ref, hw_ref, al_ref, ar_ref):
    h1 = _elu(agg_ref[...] + b1_ref[...][None, :])
    hw = jnp.dot(h1, w_ref[...], preferred_element_type=jnp.float32)
    hw_ref[...] = hw
    al_ref[...] = (hw * asrc_ref[...]).sum(axis=1, keepdims=True)
    ar_ref[...] = (hw * adst_ref[...]).sum(axis=1, keepdims=True)


def _pool_kernel(batch_ref, agg_ref, b2_ref, gsum_ref, gmax_ref, cnt_ref):
    @pl.when(pl.program_id(0) == 0)
    def _():
        gsum_ref[...] = jnp.zeros_like(gsum_ref)
        gmax_ref[...] = jnp.full_like(gmax_ref, -jnp.inf)
        cnt_ref[...] = jnp.zeros_like(cnt_ref)

    def body(i, c):
        g = batch_ref[0, 0, i]
        row = _elu(agg_ref[pl.ds(i, 1), :] + b2_ref[...][None, :])
        gsum_ref[pl.ds(g, 1), :] += row
        gmax_ref[pl.ds(g, 1), :] = jnp.maximum(gmax_ref[pl.ds(g, 1), :], row)
        cnt_ref[pl.ds(g, 1), :] += 1.0
        return c
    jax.lax.fori_loop(0, agg_ref.shape[0], body, 0)


def _head_kernel(gsum_ref, gmax_ref, cnt_ref, card_ref, gf_ref,
                 wc1_ref, bc1_ref, wc2_ref, bc2_ref, wg_ref, bg_ref,
                 wf1a_ref, wf1b_ref, wf1c_ref, wf1d_ref, bf1_ref,
                 wf2_ref, bf2_ref, out_ref):
    dot = lambda a, b: jnp.dot(a, b, preferred_element_type=jnp.float32)
    gmean = gsum_ref[...] / jnp.maximum(cnt_ref[...], 1.0)
    c = jax.nn.relu(dot(card_ref[...], wc1_ref[...]) + bc1_ref[...][None, :])
    c = jax.nn.relu(dot(c, wc2_ref[...]) + bc2_ref[...][None, :])
    g = jax.nn.relu(dot(gf_ref[...], wg_ref[...]) + bg_ref[...][None, :])
    pre = (dot(gmean, wf1a_ref[...]) + dot(gmax_ref[...], wf1b_ref[...])
           + dot(c, wf1c_ref[...]) + dot(g, wf1d_ref[...]) + bf1_ref[...][None, :])
    out = jax.nn.relu(pre)
    out_ref[...] = dot(out, wf2_ref[...]) + bf2_ref[...][None, :]


def _pad_rows(a, rows):
    return jnp.pad(a, ((0, rows - a.shape[0]),) + ((0, 0),) * (a.ndim - 1))


def kernel(x, edge_index, batch, card, genome_feat, emb, W1, a_src1, a_dst1, b1,
           W2, a_src2, a_dst2, b2, Wc1, bc1, Wc2, bc2, Wg, bg, Wf1, bf1, Wf2, bf2):
    f32 = jnp.float32
    N = x.shape[0]
    E = edge_index.shape[1]
    EMB = emb.shape[1]
    HID = W1.shape[1]
    HEADS = a_src1.shape[0]
    G = card.shape[0]

    NPAD = ((N + _BN - 1) // _BN) * _BN
    NR = NPAD // 128
    ET = E + N
    EPAD = ((ET + _BE - 1) // _BE) * _BE
    nb_n = NPAD // _BN
    nb_e = EPAD // _BE
    GP = ((G + 1 + 7) // 8) * 8

    ids = _pad_rows(x[:, 0:1].astype(jnp.int32), NPAD)[:, 0].reshape(nb_n, 1, _BN)
    loop = jnp.arange(N, dtype=jnp.int32)
    src = jnp.concatenate([edge_index[0].astype(jnp.int32), loop])
    dst = jnp.concatenate([edge_index[1].astype(jnp.int32), loop])
    pad_e = jnp.full((EPAD - ET,), N, dtype=jnp.int32)
    src = jnp.concatenate([src, pad_e]).reshape(nb_e, 1, _BE)
    dst = jnp.concatenate([dst, pad_e]).reshape(nb_e, 1, _BE)
    batch_p = jnp.concatenate(
        [batch.astype(jnp.int32), jnp.full((NPAD - N,), G, jnp.int32)]
    ).reshape(nb_n, 1, _BN)

    h0 = pl.pallas_call(
        _embed_kernel,
        grid=(nb_n,),
        in_specs=[
            pl.BlockSpec((1, 1, _BN), lambda i: (i, 0, 0), memory_space=_SMEM),
            pl.BlockSpec(emb.shape, lambda i: (0, 0)),
        ],
        out_specs=pl.BlockSpec((_BN, EMB), lambda i: (i, 0)),
        out_shape=jax.ShapeDtypeStruct((NPAD, EMB), f32),
    )(ids, emb.astype(f32))

    def dense1(h_in, W, asrc, adst):
        return pl.pallas_call(
            _dense1_kernel,
            grid=(nb_n,),
            in_specs=[
                pl.BlockSpec((_BN, h_in.shape[1]), lambda i: (i, 0)),
                pl.BlockSpec(W.shape, lambda i: (0, 0)),
                pl.BlockSpec(asrc.shape, lambda i: (0, 0)),
                pl.BlockSpec(adst.shape, lambda i: (0, 0)),
            ],
            out_specs=[
                pl.BlockSpec((_BN, W.shape[1]), lambda i: (i, 0)),
                pl.BlockSpec((_BN, asrc.shape[0]), lambda i: (i, 0)),
                pl.BlockSpec((_BN, asrc.shape[0]), lambda i: (i, 0)),
            ],
            out_shape=[
                jax.ShapeDtypeStruct((NPAD, W.shape[1]), f32),
                jax.ShapeDtypeStruct((NPAD, asrc.shape[0]), f32),
                jax.ShapeDtypeStruct((NPAD, asrc.shape[0]), f32),
            ],
        )(h_in, W, asrc, adst)

    def den_pass(srcb, dstb, alT, arT):
        H = alT.shape[0]
        return pl.pallas_call(
            _den_kernel,
            grid=(nb_e,),
            in_specs=[
                pl.BlockSpec((1, 1, _BE), lambda i: (i, 0, 0), memory_space=_SMEM),
                pl.BlockSpec((1, 1, _BE), lambda i: (i, 0, 0), memory_space=_SMEM),
                pl.BlockSpec(alT.shape, lambda i: (0, 0, 0)),
                pl.BlockSpec(arT.shape, lambda i: (0, 0, 0)),
            ],
            out_specs=pl.BlockSpec((H, NR, 128), lambda i: (0, 0, 0)),
            out_shape=jax.ShapeDtypeStruct((H, NR, 128), f32),
        )(srcb, dstb, alT, arT)

    def agg_pass(srcb, dstb, alT, arT, den, hw, n_chunk):
        CW = HID // n_chunk
        heads = alT.shape[0]
        hpc = heads // n_chunk if heads >= n_chunk else 1
        outs = []
        for c in range(n_chunk):
            body = functools.partial(
                _agg_kernel, c * hpc if heads >= n_chunk else 0, hpc)
            hw_c = jax.lax.slice(hw, (0, c * CW), (NPAD, (c + 1) * CW))
            outs.append(pl.pallas_call(
                body,
                grid=(nb_e,),
                in_specs=[
                    pl.BlockSpec((1, 1, _BE), lambda i: (i, 0, 0), memory_space=_SMEM),
                    pl.BlockSpec((1, 1, _BE), lambda i: (i, 0, 0), memory_space=_SMEM),
                    pl.BlockSpec(alT.shape, lambda i: (0, 0, 0)),
                    pl.BlockSpec(arT.shape, lambda i: (0, 0, 0)),
                    pl.BlockSpec(den.shape, lambda i: (0, 0, 0)),
                    pl.BlockSpec((NPAD, CW), lambda i: (0, 0)),
                ],
                out_specs=pl.BlockSpec((NPAD, CW), lambda i: (0, 0)),
                out_shape=jax.ShapeDtypeStruct((NPAD, CW), f32),
            )(srcb, dstb, alT, arT, den, hw_c))
        return jnp.concatenate(outs, axis=1)

    hw1, al1, ar1 = dense1(h0, W1, a_src1, a_dst1)
    al1T = al1.T.reshape(HEADS, NR, 128)
    ar1T = ar1.T.reshape(HEADS, NR, 128)
    den1 = den_pass(src, dst, al1T, ar1T)
    agg1 = agg_pass(src, dst, al1T, ar1T, den1, hw1, 2)

    hw2, al2, ar2 = pl.pallas_call(
        _dense2_kernel,
        grid=(nb_n,),
        in_specs=[
            pl.BlockSpec((_BN, HID), lambda i: (i, 0)),
            pl.BlockSpec(b1.shape, lambda i: (0,)),
            pl.BlockSpec(W2.shape, lambda i: (0, 0)),
            pl.BlockSpec(a_src2.shape, lambda i: (0, 0)),
            pl.BlockSpec(a_dst2.shape, lambda i: (0, 0)),
        ],
        out_specs=[
            pl.BlockSpec((_BN, HID), lambda i: (i, 0)),
            pl.BlockSpec((_BN, 1), lambda i: (i, 0)),
            pl.BlockSpec((_BN, 1), lambda i: (i, 0)),
        ],
        out_shape=[
            jax.ShapeDtypeStruct((NPAD, HID), f32),
            jax.ShapeDtypeStruct((NPAD, 1), f32),
            jax.ShapeDtypeStruct((NPAD, 1), f32),
        ],
    )(agg1, b1, W2, a_src2, a_dst2)
    al2T = al2.T.reshape(1, NR, 128)
    ar2T = ar2.T.reshape(1, NR, 128)
    den2 = den_pass(src, dst, al2T, ar2T)
    agg2 = agg_pass(src, dst, al2T, ar2T, den2, hw2, 2)

    gsum, gmax, cnt = pl.pallas_call(
        _pool_kernel,
        grid=(nb_n,),
        in_specs=[
            pl.BlockSpec((1, 1, _BN), lambda i: (i, 0, 0), memory_space=_SMEM),
            pl.BlockSpec((_BN, HID), lambda i: (i, 0)),
            pl.BlockSpec(b2.shape, lambda i: (0,)),
        ],
        out_specs=[
            pl.BlockSpec((GP, HID), lambda i: (0, 0)),
            pl.BlockSpec((GP, HID), lambda i: (0, 0)),
            pl.BlockSpec((GP, HID), lambda i: (0, 0)),
        ],
        out_shape=[
            jax.ShapeDtypeStruct((GP, HID), f32),
            jax.ShapeDtypeStruct((GP, HID), f32),
            jax.ShapeDtypeStruct((GP, HID), f32),
        ],
    )(batch_p, agg2, b2)

    card_p = jnp.pad(card.astype(f32), ((0, 0), (0, 1)))
    Wc1_p = jnp.pad(Wc1, ((0, 1), (0, 0)))
    gf_p = jnp.pad(genome_feat.astype(f32), ((0, 0), (0, 5)))
    Wg_p = jnp.pad(Wg, ((0, 5), (0, 0)))
    OW = ((Wf2.shape[1] + 127) // 128) * 128
    Wf2_p = jnp.pad(Wf2, ((0, 0), (0, OW - Wf2.shape[1])))
    bf2_p = jnp.pad(bf2, ((0, OW - Wf2.shape[1]),))
    Wf1a = Wf1[:HID]
    Wf1b = Wf1[HID:2 * HID]
    Wf1c = Wf1[2 * HID:2 * HID + Wc2.shape[1]]
    Wf1d = Wf1[2 * HID + Wc2.shape[1]:]

    full = lambda a: pl.BlockSpec(a.shape, lambda: tuple(0 for _ in a.shape))
    args = [gsum[:G], gmax[:G], cnt[:G], card_p, gf_p, Wc1_p, bc1, Wc2, bc2,
            Wg_p, bg, Wf1a, Wf1b, Wf1c, Wf1d, bf1, Wf2_p, bf2_p]
    out = pl.pallas_call(
        _head_kernel,
        in_specs=[full(a) for a in args],
        out_specs=pl.BlockSpec((G, OW), lambda: (0, 0)),
        out_shape=jax.ShapeDtypeStruct((G, OW), f32),
    )(*args)
    return out[:, :Wf2.shape[1]]
